# baseline (speedup 1.0000x reference)
import functools

import jax
import jax.numpy as jnp
from jax import lax
from jax.experimental import pallas as pl
from jax.experimental.pallas import tpu as pltpu
from jax.experimental.pallas import tpu_sc as plsc

_C = 19
_KEEP = 512
_RB = 32
_PINF = 0x7F800000


def _stage1_body(inp_ref, tgt_ref, dist_ref, nll_ref, mx_ref):
    t = tgt_ref[0]
    x = inp_ref[0, 0]
    s_exp = jnp.exp(x)
    s_abs = jnp.abs(x)
    xt = jnp.where(t == 0, x, 0.0)
    for c in range(1, _C):
        x = inp_ref[0, c]
        s_exp = s_exp + jnp.exp(x)
        s_abs = s_abs + jnp.abs(x)
        xt = jnp.where(t == c, x, xt)
    dist = s_abs - jnp.abs(xt) + jnp.abs(xt - 1.0)
    dist_ref[0] = dist
    nll_ref[0] = jnp.log(s_exp) - xt
    mx_ref[0] = jnp.max(dist.reshape(_RB, 32, 16), axis=-1)


def _sc_batch(b, mx_hbm, distr_hbm, nllr_hbm, out_hbm, mx_v, l3_v, cand_v,
              rowidx_v, r128idx_v, rows128_v, ebits_v, eidx_v, gr128_v,
              goff_v, qr128_v, qoff_v, outb_v):
    iota = lax.iota(jnp.int32, 16)
    zeros_i = jnp.zeros((16,), jnp.int32)

    def _zero(ref, n):
        @pl.loop(0, n, step=16)
        def _(i):
            ref[pl.ds(i, 16)] = zeros_i

    _zero(cand_v, 16384)
    _zero(ebits_v, 16384)
    _zero(rowidx_v, 1024)
    _zero(r128idx_v, 1024)
    _zero(gr128_v, 1024)
    _zero(goff_v, 1024)
    _zero(qr128_v, 1024)
    _zero(qoff_v, 1024)

    pltpu.sync_copy(mx_hbm.at[b], mx_v)

    @pl.loop(0, 1024, step=16)
    def _(j):
        idx0 = j * 16 + iota * 16
        acc = plsc.load_gather(mx_v, [idx0])
        for k in range(1, 16):
            acc = jnp.maximum(acc, plsc.load_gather(mx_v, [idx0 + k]))
        l3_v[pl.ds(j, 16)] = acc

    def _count_gt(ref, is_f32, nslice, mid):
        def step(i, cnt):
            v = ref[pl.ds(i * 16, 16)]
            bits = plsc.bitcast(v, jnp.int32) if is_f32 else v
            c = plsc.all_reduce_population_count(bits > mid)
            return cnt + c[0]

        return lax.fori_loop(0, nslice, step, jnp.int32(0))

    def _kth_bits(ref, is_f32, nslice):
        def bis(i, lohi):
            lo, hi = lohi
            mid = lo + (hi - lo) // 2
            below = _count_gt(ref, is_f32, nslice, mid) < _KEEP
            return (jnp.where(below, lo, mid), jnp.where(below, mid, hi))

        _, hi = lax.fori_loop(
            0, 31, bis, (jnp.int32(-1), jnp.int32(_PINF)))
        return hi

    f3 = _kth_bits(l3_v, True, jnp.int32(64))

    def cstep(i, ptr):
        bits = plsc.bitcast(mx_v[pl.ds(i * 16, 16)], jnp.int32)
        mask = bits >= f3
        plsc.store_compressed(cand_v.at[pl.ds(ptr, 16)], bits, mask=mask)
        c = plsc.all_reduce_population_count(mask)
        return jnp.minimum(ptr + c[0], 16384 - 16)

    ncand = lax.fori_loop(0, 1024, cstep, jnp.int32(0))
    f2 = _kth_bits(cand_v, False, (ncand + 15) // 16)

    rowbase = b * 16384

    def rstep(i, ptr):
        bits = plsc.bitcast(mx_v[pl.ds(i * 16, 16)], jnp.int32)
        mask = bits >= f2
        gids = rowbase + i * 16 + iota
        plsc.store_compressed(rowidx_v.at[pl.ds(ptr, 16)], gids, mask=mask)
        plsc.store_compressed(r128idx_v.at[pl.ds(ptr, 16)], gids >> 3,
                              mask=mask)
        c = plsc.all_reduce_population_count(mask)
        return jnp.minimum(ptr + c[0], 1024 - 16)

    rowcnt = lax.fori_loop(0, 1024, rstep, jnp.int32(0))

    def echunk(k, eptr):
        pltpu.sync_copy(distr_hbm.at[r128idx_v.at[pl.ds(k * 128, 128)]],
                        rows128_v)

        def estep(j, eptr):
            base = k * 128 + j * 16
            rvec = rowidx_v[pl.ds(base, 16)]
            for t in range(16):
                rloc = base + t
                cols = (rvec[t] & 7) * 16 + iota
                vals = plsc.load_gather(
                    rows128_v, [jnp.full((16,), j * 16 + t), cols])
                bits = plsc.bitcast(vals, jnp.int32)
                mask = (bits >= f2) & (rloc < rowcnt)
                plsc.store_compressed(ebits_v.at[pl.ds(eptr, 16)], bits,
                                      mask=mask)
                plsc.store_compressed(eidx_v.at[pl.ds(eptr, 16)],
                                      rvec[t] * 16 + iota, mask=mask)
                c = plsc.all_reduce_population_count(mask)
                eptr = jnp.minimum(eptr + c[0], 16384 - 16)
            return eptr

        return lax.fori_loop(0, 8, estep, eptr)

    ecnt = lax.fori_loop(0, (rowcnt + 127) // 128, echunk, jnp.int32(0))

    thr = _kth_bits(ebits_v, False, (ecnt + 15) // 16)

    def sstep(i, carry):
        m, e, gptr, qptr = carry
        bits = ebits_v[pl.ds(i * 16, 16)]
        eidx = eidx_v[pl.ds(i * 16, 16)]
        gt = bits > thr
        eq = bits == thr
        r128v = eidx >> 7
        offv = eidx & 127
        plsc.store_compressed(gr128_v.at[pl.ds(gptr, 16)], r128v, mask=gt)
        plsc.store_compressed(goff_v.at[pl.ds(gptr, 16)], offv, mask=gt)
        plsc.store_compressed(qr128_v.at[pl.ds(qptr, 16)], r128v, mask=eq)
        plsc.store_compressed(qoff_v.at[pl.ds(qptr, 16)], offv, mask=eq)
        gc = plsc.all_reduce_population_count(gt)
        qc = plsc.all_reduce_population_count(eq)
        return (m + gc[0], e + qc[0],
                jnp.minimum(gptr + gc[0], 1024 - 16),
                jnp.minimum(qptr + qc[0], 1024 - 16))

    z = jnp.int32(0)
    m, e, _, _ = lax.fori_loop(0, (ecnt + 15) // 16, sstep, (z, z, z, z))
    e = jnp.minimum(e, 1008)

    def _gather_sum(r128_ref, off_ref, n):
        def chunk(k, acc):
            pltpu.sync_copy(nllr_hbm.at[r128_ref.at[pl.ds(k * 128, 128)]],
                            rows128_v)

            def ss(i, acc):
                pos = k * 128 + i * 16 + iota
                offs = off_ref[pl.ds(k * 128 + i * 16, 16)]
                vals = plsc.load_gather(rows128_v, [i * 16 + iota, offs])
                return acc + jnp.where(pos < n, vals, 0.0)

            return lax.fori_loop(0, 8, ss, acc)

        acc = lax.fori_loop(0, (n + 127) // 128, chunk,
                            jnp.zeros((16,), jnp.float32))
        return jnp.sum(acc, axis=0)

    s_gt = _gather_sum(gr128_v, goff_v, m)
    s_eq = _gather_sum(qr128_v, qoff_v, e)

    stats = jnp.where(iota == 0, s_gt, 0.0)
    stats = jnp.where(iota == 1, s_eq, stats)
    stats = jnp.where(iota == 2, m.astype(jnp.float32), stats)
    stats = jnp.where(iota == 3, e.astype(jnp.float32), stats)
    outb_v[...] = stats
    pltpu.sync_copy(outb_v, out_hbm.at[b])


def _sc_body(mx_hbm, distr_hbm, nllr_hbm, out_hbm, *scratch):
    cid = lax.axis_index("core")
    sid = lax.axis_index("subcore")

    @pl.when(sid < 4)
    def _():
        _sc_batch(cid * 4 + sid, mx_hbm, distr_hbm, nllr_hbm, out_hbm,
                  *scratch)


def _sc_select(mxr, distr, nllr):
    mesh = plsc.VectorSubcoreMesh(core_axis_name="core",
                                  subcore_axis_name="subcore",
                                  num_cores=2, num_subcores=16)
    return pl.kernel(
        _sc_body,
        out_type=jax.ShapeDtypeStruct((8, 16), jnp.float32),
        mesh=mesh,
        compiler_params=pltpu.CompilerParams(needs_layout_passes=False),
        scratch_types=[
            pltpu.VMEM((16384,), jnp.float32),
            pltpu.VMEM((1024,), jnp.float32),
            pltpu.VMEM((16384,), jnp.int32),
            pltpu.VMEM((1024,), jnp.int32),
            pltpu.VMEM((1024,), jnp.int32),
            pltpu.VMEM((128, 128), jnp.float32),
            pltpu.VMEM((16384,), jnp.int32),
            pltpu.VMEM((16384,), jnp.int32),
            pltpu.VMEM((1024,), jnp.int32),
            pltpu.VMEM((1024,), jnp.int32),
            pltpu.VMEM((1024,), jnp.int32),
            pltpu.VMEM((1024,), jnp.int32),
            pltpu.VMEM((16,), jnp.float32),
        ],
    )(mxr, distr, nllr)


def kernel(inputs, targets):
    B, C, H, W = inputs.shape
    grid1 = (B, H // _RB)
    dist, nll, mx = pl.pallas_call(
        _stage1_body,
        grid=grid1,
        in_specs=[
            pl.BlockSpec((1, C, _RB, W), lambda b, i: (b, 0, i, 0)),
            pl.BlockSpec((1, _RB, W), lambda b, i: (b, i, 0)),
        ],
        out_specs=[
            pl.BlockSpec((1, _RB, W), lambda b, i: (b, i, 0)),
            pl.BlockSpec((1, _RB, W), lambda b, i: (b, i, 0)),
            pl.BlockSpec((1, _RB, 32), lambda b, i: (b, i, 0)),
        ],
        out_shape=[
            jax.ShapeDtypeStruct((B, H, W), jnp.float32),
            jax.ShapeDtypeStruct((B, H, W), jnp.float32),
            jax.ShapeDtypeStruct((B, H, 32), jnp.float32),
        ],
    )(inputs, targets)

    nrow128 = B * H * 4
    out = _sc_select(mx.reshape(B, H * 32),
                     dist.reshape(nrow128, 128),
                     nll.reshape(nrow128, 128))
    s_gt, s_eq, m, e = out[:, 0], out[:, 1], out[:, 2], out[:, 3]
    contrib = s_gt + (_KEEP - m) * s_eq / e
    return jnp.sum(contrib) / (B * _KEEP)


# speedup vs baseline: 1.0972x; 1.0972x over previous
import functools

import jax
import jax.numpy as jnp
from jax import lax
from jax.experimental import pallas as pl
from jax.experimental.pallas import tpu as pltpu
from jax.experimental.pallas import tpu_sc as plsc

_C = 19
_KEEP = 512
_RB = 32
_PINF = 0x7F800000


def _stage1_body(inp_ref, tgt_ref, dist_ref, nll_ref, mx_ref):
    t = tgt_ref[0]
    x = inp_ref[0, 0]
    s_exp = jnp.exp(x)
    s_abs = jnp.abs(x)
    xt = jnp.where(t == 0, x, 0.0)
    for c in range(1, _C):
        x = inp_ref[0, c]
        s_exp = s_exp + jnp.exp(x)
        s_abs = s_abs + jnp.abs(x)
        xt = jnp.where(t == c, x, xt)
    dist = s_abs - jnp.abs(xt) + jnp.abs(xt - 1.0)
    dist_ref[0] = dist
    nll_ref[0] = jnp.log(s_exp) - xt
    mx_ref[0] = jnp.max(dist.reshape(_RB, 32, 16), axis=-1)


def _sc_batch(b, mx_hbm, distr_hbm, nllr_hbm, out_hbm, mx_v, l3_v, cand_v,
              rowidx_v, r128idx_v, rows128_v, ebits_v, eidx_v, gr128_v,
              goff_v, qr128_v, qoff_v, outb_v):
    iota = lax.iota(jnp.int32, 16)
    zeros_i = jnp.zeros((16,), jnp.int32)

    def _zero(ref, n):
        @pl.loop(0, n, step=128)
        def _(i):
            for u in range(8):
                ref[pl.ds(i + u * 16, 16)] = zeros_i

    _zero(cand_v, 16384)
    _zero(ebits_v, 16384)
    _zero(rowidx_v, 1024)
    _zero(r128idx_v, 1024)
    _zero(gr128_v, 1024)
    _zero(goff_v, 1024)
    _zero(qr128_v, 1024)
    _zero(qoff_v, 1024)

    pltpu.sync_copy(mx_hbm.at[b], mx_v)

    @pl.loop(0, 1024, step=16)
    def _(j):
        idx0 = j * 16 + iota * 16
        acc = plsc.load_gather(mx_v, [idx0])
        for k in range(1, 16):
            acc = jnp.maximum(acc, plsc.load_gather(mx_v, [idx0 + k]))
        l3_v[pl.ds(j, 16)] = acc

    def _count_gt(ref, is_f32, nslice128, mid):
        def step(i, cv):
            for u in range(8):
                v = ref[pl.ds(i * 128 + u * 16, 16)]
                bits = plsc.bitcast(v, jnp.int32) if is_f32 else v
                cv = cv + (bits > mid).astype(jnp.int32)
            return cv

        cv = lax.fori_loop(0, nslice128, step, jnp.zeros((16,), jnp.int32))
        return jnp.sum(cv, axis=0)

    def _kth_bits(ref, is_f32, nslice128, lo0, hi0):
        def cond(lohi):
            lo, hi = lohi
            return hi - lo > 1

        def bis(lohi):
            lo, hi = lohi
            mid = lo + (hi - lo) // 2
            below = _count_gt(ref, is_f32, nslice128, mid) < _KEEP
            return (jnp.where(below, lo, mid), jnp.where(below, mid, hi))

        _, hi = lax.while_loop(cond, bis, (lo0, hi0))
        return hi

    def mmstep(i, mv):
        return jnp.maximum(mv, l3_v[pl.ds(i * 16, 16)])

    mxall = lax.fori_loop(0, 64, mmstep, jnp.zeros((16,), jnp.float32))
    maxbits = jnp.max(plsc.bitcast(mxall, jnp.int32), axis=0)

    f3 = _kth_bits(l3_v, True, jnp.int32(8), jnp.int32(-1), maxbits + 1)

    def cstep(i, ptr):
        for u in range(4):
            bits = plsc.bitcast(mx_v[pl.ds(i * 64 + u * 16, 16)], jnp.int32)
            mask = bits >= f3
            plsc.store_compressed(cand_v.at[pl.ds(ptr, 16)], bits, mask=mask)
            c = plsc.all_reduce_population_count(mask)
            ptr = jnp.minimum(ptr + c[0], 16384 - 16)
        return ptr

    ncand = lax.fori_loop(0, 256, cstep, jnp.int32(0))
    f2 = _kth_bits(cand_v, False, (ncand + 127) // 128, f3 - 1,
                   maxbits + 1)

    rowbase = b * 16384

    def rstep(i, ptr):
        for u in range(4):
            bits = plsc.bitcast(mx_v[pl.ds(i * 64 + u * 16, 16)], jnp.int32)
            mask = bits >= f2
            gids = rowbase + i * 64 + u * 16 + iota
            plsc.store_compressed(rowidx_v.at[pl.ds(ptr, 16)], gids,
                                  mask=mask)
            plsc.store_compressed(r128idx_v.at[pl.ds(ptr, 16)], gids >> 3,
                                  mask=mask)
            c = plsc.all_reduce_population_count(mask)
            ptr = jnp.minimum(ptr + c[0], 1024 - 16)
        return ptr

    rowcnt = lax.fori_loop(0, 256, rstep, jnp.int32(0))

    def echunk(k, eptr):
        pltpu.sync_copy(distr_hbm.at[r128idx_v.at[pl.ds(k * 128, 128)]],
                        rows128_v)

        def estep(j, eptr):
            base = k * 128 + j * 16
            for t in range(16):
                rloc = base + t
                rv = plsc.load_gather(rowidx_v, [jnp.full((16,), rloc)])
                cols = (rv & 7) * 16 + iota
                vals = plsc.load_gather(
                    rows128_v, [jnp.full((16,), j * 16 + t), cols])
                bits = plsc.bitcast(vals, jnp.int32)
                mask = (bits >= f2) & (rloc < rowcnt)
                plsc.store_compressed(ebits_v.at[pl.ds(eptr, 16)], bits,
                                      mask=mask)
                plsc.store_compressed(eidx_v.at[pl.ds(eptr, 16)],
                                      rv * 16 + iota, mask=mask)
                c = plsc.all_reduce_population_count(mask)
                eptr = jnp.minimum(eptr + c[0], 16384 - 16)
            return eptr

        return lax.fori_loop(0, 8, estep, eptr)

    ecnt = lax.fori_loop(0, (rowcnt + 127) // 128, echunk, jnp.int32(0))

    thr = _kth_bits(ebits_v, False, (ecnt + 127) // 128, f2 - 1,
                    maxbits + 1)

    def sstep(i, carry):
        m, e, gptr, qptr = carry
        bits = ebits_v[pl.ds(i * 16, 16)]
        eidx = eidx_v[pl.ds(i * 16, 16)]
        gt = bits > thr
        eq = bits == thr
        r128v = eidx >> 7
        offv = eidx & 127
        plsc.store_compressed(gr128_v.at[pl.ds(gptr, 16)], r128v, mask=gt)
        plsc.store_compressed(goff_v.at[pl.ds(gptr, 16)], offv, mask=gt)
        plsc.store_compressed(qr128_v.at[pl.ds(qptr, 16)], r128v, mask=eq)
        plsc.store_compressed(qoff_v.at[pl.ds(qptr, 16)], offv, mask=eq)
        gc = plsc.all_reduce_population_count(gt)
        qc = plsc.all_reduce_population_count(eq)
        return (m + gc[0], e + qc[0],
                jnp.minimum(gptr + gc[0], 1024 - 16),
                jnp.minimum(qptr + qc[0], 1024 - 16))

    z = jnp.int32(0)
    m, e, _, _ = lax.fori_loop(0, (ecnt + 15) // 16, sstep, (z, z, z, z))
    e = jnp.minimum(e, 1008)

    def _gather_sum(r128_ref, off_ref, n):
        def chunk(k, acc):
            pltpu.sync_copy(nllr_hbm.at[r128_ref.at[pl.ds(k * 128, 128)]],
                            rows128_v)

            def ss(i, acc):
                pos = k * 128 + i * 16 + iota
                offs = off_ref[pl.ds(k * 128 + i * 16, 16)]
                vals = plsc.load_gather(rows128_v, [i * 16 + iota, offs])
                return acc + jnp.where(pos < n, vals, 0.0)

            return lax.fori_loop(0, 8, ss, acc)

        acc = lax.fori_loop(0, (n + 127) // 128, chunk,
                            jnp.zeros((16,), jnp.float32))
        return jnp.sum(acc, axis=0)

    s_gt = _gather_sum(gr128_v, goff_v, m)
    s_eq = _gather_sum(qr128_v, qoff_v, e)

    stats = jnp.where(iota == 0, s_gt, 0.0)
    stats = jnp.where(iota == 1, s_eq, stats)
    stats = jnp.where(iota == 2, m.astype(jnp.float32), stats)
    stats = jnp.where(iota == 3, e.astype(jnp.float32), stats)
    outb_v[...] = stats
    pltpu.sync_copy(outb_v, out_hbm.at[b])


def _sc_body(mx_hbm, distr_hbm, nllr_hbm, out_hbm, *scratch):
    cid = lax.axis_index("core")
    sid = lax.axis_index("subcore")

    @pl.when(sid < 4)
    def _():
        _sc_batch(cid * 4 + sid, mx_hbm, distr_hbm, nllr_hbm, out_hbm,
                  *scratch)


def _sc_select(mxr, distr, nllr):
    mesh = plsc.VectorSubcoreMesh(core_axis_name="core",
                                  subcore_axis_name="subcore",
                                  num_cores=2, num_subcores=16)
    return pl.kernel(
        _sc_body,
        out_type=jax.ShapeDtypeStruct((8, 16), jnp.float32),
        mesh=mesh,
        compiler_params=pltpu.CompilerParams(needs_layout_passes=False),
        scratch_types=[
            pltpu.VMEM((16384,), jnp.float32),
            pltpu.VMEM((1024,), jnp.float32),
            pltpu.VMEM((16384,), jnp.int32),
            pltpu.VMEM((1024,), jnp.int32),
            pltpu.VMEM((1024,), jnp.int32),
            pltpu.VMEM((128, 128), jnp.float32),
            pltpu.VMEM((16384,), jnp.int32),
            pltpu.VMEM((16384,), jnp.int32),
            pltpu.VMEM((1024,), jnp.int32),
            pltpu.VMEM((1024,), jnp.int32),
            pltpu.VMEM((1024,), jnp.int32),
            pltpu.VMEM((1024,), jnp.int32),
            pltpu.VMEM((16,), jnp.float32),
        ],
    )(mxr, distr, nllr)


def kernel(inputs, targets):
    B, C, H, W = inputs.shape
    grid1 = (B, H // _RB)
    dist, nll, mx = pl.pallas_call(
        _stage1_body,
        grid=grid1,
        in_specs=[
            pl.BlockSpec((1, C, _RB, W), lambda b, i: (b, 0, i, 0)),
            pl.BlockSpec((1, _RB, W), lambda b, i: (b, i, 0)),
        ],
        out_specs=[
            pl.BlockSpec((1, _RB, W), lambda b, i: (b, i, 0)),
            pl.BlockSpec((1, _RB, W), lambda b, i: (b, i, 0)),
            pl.BlockSpec((1, _RB, 32), lambda b, i: (b, i, 0)),
        ],
        out_shape=[
            jax.ShapeDtypeStruct((B, H, W), jnp.float32),
            jax.ShapeDtypeStruct((B, H, W), jnp.float32),
            jax.ShapeDtypeStruct((B, H, 32), jnp.float32),
        ],
    )(inputs, targets)

    nrow128 = B * H * 4
    out = _sc_select(mx.reshape(B, H * 32),
                     dist.reshape(nrow128, 128),
                     nll.reshape(nrow128, 128))
    s_gt, s_eq, m, e = out[:, 0], out[:, 1], out[:, 2], out[:, 3]
    contrib = s_gt + (_KEEP - m) * s_eq / e
    return jnp.sum(contrib) / (B * _KEEP)


# speedup vs baseline: 1.2958x; 1.1810x over previous
import functools

import jax
import jax.numpy as jnp
from jax import lax
from jax.experimental import pallas as pl
from jax.experimental.pallas import tpu as pltpu
from jax.experimental.pallas import tpu_sc as plsc

_C = 19
_KEEP = 512
_RB = 64
_PINF = 0x7F800000


def _stage1_body(inp_ref, tgt_ref, dist_ref, nll_ref, mx_ref):
    t = tgt_ref[0]
    x = inp_ref[0, 0]
    s_exp = jnp.exp(x)
    s_abs = jnp.abs(x)
    xt = jnp.where(t == 0, x, 0.0)
    for c in range(1, _C):
        x = inp_ref[0, c]
        s_exp = s_exp + jnp.exp(x)
        s_abs = s_abs + jnp.abs(x)
        xt = jnp.where(t == c, x, xt)
    dist = s_abs - jnp.abs(xt) + jnp.abs(xt - 1.0)
    dist_ref[0] = dist
    nll_ref[0] = jnp.log(s_exp) - xt
    mx_ref[0] = jnp.max(dist.reshape(_RB, 32, 16), axis=-1)


def _sc_batch(b, mx_hbm, distr_hbm, nllr_hbm, out_hbm, mx_v, l3_v,
              rowidx_v, r128idx_v, rows512_v, ebits_v, eidx_v, gr128_v,
              goff_v, qr128_v, qoff_v, outb_v, dsem):
    iota = lax.iota(jnp.int32, 16)
    zeros_i = jnp.zeros((16,), jnp.int32)

    def _zero(ref, n):
        @pl.loop(0, n, step=128)
        def _(i):
            for u in range(8):
                ref[pl.ds(i + u * 16, 16)] = zeros_i

    _zero(ebits_v, 16384)
    _zero(rowidx_v, 1024)
    _zero(r128idx_v, 1024)
    _zero(gr128_v, 1024)
    _zero(goff_v, 1024)
    _zero(qr128_v, 1024)
    _zero(qoff_v, 1024)

    pltpu.sync_copy(mx_hbm.at[b], mx_v)

    @pl.loop(0, 1024, step=16)
    def _(j):
        idx0 = j * 16 + iota * 16
        acc = plsc.load_gather(mx_v, [idx0])
        for k in range(1, 16):
            acc = jnp.maximum(acc, plsc.load_gather(mx_v, [idx0 + k]))
        l3_v[pl.ds(j, 16)] = acc

    def _count_gt(ref, is_f32, nslice128, mid):
        def step(i, cv):
            for u in range(8):
                v = ref[pl.ds(i * 128 + u * 16, 16)]
                bits = plsc.bitcast(v, jnp.int32) if is_f32 else v
                cv = cv + (bits > mid).astype(jnp.int32)
            return cv

        cv = lax.fori_loop(0, nslice128, step, jnp.zeros((16,), jnp.int32))
        return jnp.sum(cv, axis=0)

    def _kth_bits(ref, is_f32, nslice128, lo0, hi0):
        def cond(lohi):
            lo, hi = lohi
            return hi - lo > 1

        def bis(lohi):
            lo, hi = lohi
            mid = lo + (hi - lo) // 2
            below = _count_gt(ref, is_f32, nslice128, mid) < _KEEP
            return (jnp.where(below, lo, mid), jnp.where(below, mid, hi))

        _, hi = lax.while_loop(cond, bis, (lo0, hi0))
        return hi

    def mmstep(i, mv):
        return jnp.maximum(mv, l3_v[pl.ds(i * 16, 16)])

    mxall = lax.fori_loop(0, 64, mmstep, jnp.zeros((16,), jnp.float32))
    maxbits = jnp.max(plsc.bitcast(mxall, jnp.int32), axis=0)

    f3 = _kth_bits(l3_v, True, jnp.int32(8), jnp.int32(-1), maxbits + 1)

    def cstep(i, ptr):
        for u in range(4):
            bits = plsc.bitcast(mx_v[pl.ds(i * 64 + u * 16, 16)], jnp.int32)
            mask = bits >= f3
            plsc.store_compressed(ebits_v.at[pl.ds(ptr, 16)], bits, mask=mask)
            c = plsc.all_reduce_population_count(mask)
            ptr = jnp.minimum(ptr + c[0], 16384 - 16)
        return ptr

    ncand = lax.fori_loop(0, 256, cstep, jnp.int32(0))
    f2 = _kth_bits(ebits_v, False, (ncand + 127) // 128, f3 - 1,
                   maxbits + 1)

    _zero(ebits_v, 16384)

    rowbase = b * 16384

    def rstep(i, ptr):
        for u in range(4):
            bits = plsc.bitcast(mx_v[pl.ds(i * 64 + u * 16, 16)], jnp.int32)
            mask = bits >= f2
            gids = rowbase + i * 64 + u * 16 + iota
            plsc.store_compressed(rowidx_v.at[pl.ds(ptr, 16)], gids,
                                  mask=mask)
            plsc.store_compressed(r128idx_v.at[pl.ds(ptr, 16)], gids >> 3,
                                  mask=mask)
            c = plsc.all_reduce_population_count(mask)
            ptr = jnp.minimum(ptr + c[0], 1024 - 16)
        return ptr

    rowcnt = lax.fori_loop(0, 256, rstep, jnp.int32(0))

    copies = [
        pltpu.async_copy(
            distr_hbm.at[r128idx_v.at[pl.ds(k * 128, 128)]],
            rows512_v.at[pl.ds(k * 128, 128), :], dsem)
        for k in range(4)
    ]
    for cp in copies:
        cp.wait()
    rowlim = jnp.minimum(rowcnt, 512)

    def estep(j, eptr):
        for t in range(16):
            rloc = j * 16 + t
            rv = plsc.load_gather(rowidx_v, [jnp.full((16,), rloc)])
            cols = (rv & 7) * 16 + iota
            vals = plsc.load_gather(rows512_v,
                                    [jnp.full((16,), rloc), cols])
            bits = plsc.bitcast(vals, jnp.int32)
            mask = (bits >= f2) & (rloc < rowlim)
            plsc.store_compressed(ebits_v.at[pl.ds(eptr, 16)], bits,
                                  mask=mask)
            plsc.store_compressed(eidx_v.at[pl.ds(eptr, 16)],
                                  rv * 16 + iota, mask=mask)
            c = plsc.all_reduce_population_count(mask)
            eptr = jnp.minimum(eptr + c[0], 16384 - 16)
        return eptr

    ecnt = lax.fori_loop(0, (rowlim + 15) // 16, estep, jnp.int32(0))

    thr = _kth_bits(ebits_v, False, (ecnt + 127) // 128, f2 - 1,
                    maxbits + 1)

    def sstep(i, carry):
        m, e, gptr, qptr = carry
        bits = ebits_v[pl.ds(i * 16, 16)]
        eidx = eidx_v[pl.ds(i * 16, 16)]
        gt = bits > thr
        eq = bits == thr
        r128v = eidx >> 7
        offv = eidx & 127
        plsc.store_compressed(gr128_v.at[pl.ds(gptr, 16)], r128v, mask=gt)
        plsc.store_compressed(goff_v.at[pl.ds(gptr, 16)], offv, mask=gt)
        plsc.store_compressed(qr128_v.at[pl.ds(qptr, 16)], r128v, mask=eq)
        plsc.store_compressed(qoff_v.at[pl.ds(qptr, 16)], offv, mask=eq)
        gc = plsc.all_reduce_population_count(gt)
        qc = plsc.all_reduce_population_count(eq)
        return (m + gc[0], e + qc[0],
                jnp.minimum(gptr + gc[0], 1024 - 16),
                jnp.minimum(qptr + qc[0], 1024 - 16))

    z = jnp.int32(0)
    m, e, _, _ = lax.fori_loop(0, (ecnt + 15) // 16, sstep, (z, z, z, z))

    def _gather_sum(r128_ref, off_ref, n, nchunk):
        copies = [
            pltpu.async_copy(
                nllr_hbm.at[r128_ref.at[pl.ds(k * 128, 128)]],
                rows512_v.at[pl.ds(k * 128, 128), :], dsem)
            for k in range(nchunk)
        ]
        for cp in copies:
            cp.wait()

        def ss(i, acc):
            pos = i * 16 + iota
            offs = off_ref[pl.ds(i * 16, 16)]
            vals = plsc.load_gather(rows512_v, [pos, offs])
            return acc + jnp.where(pos < n, vals, 0.0)

        acc = lax.fori_loop(0, (n + 15) // 16, ss,
                            jnp.zeros((16,), jnp.float32))
        return jnp.sum(acc, axis=0)

    s_gt = _gather_sum(gr128_v, goff_v, m, 4)
    e = jnp.minimum(e, 128)
    s_eq = _gather_sum(qr128_v, qoff_v, e, 1)

    stats = jnp.where(iota == 0, s_gt, 0.0)
    stats = jnp.where(iota == 1, s_eq, stats)
    stats = jnp.where(iota == 2, m.astype(jnp.float32), stats)
    stats = jnp.where(iota == 3, e.astype(jnp.float32), stats)
    outb_v[...] = stats
    pltpu.sync_copy(outb_v, out_hbm.at[b])


def _sc_body(mx_hbm, distr_hbm, nllr_hbm, out_hbm, *scratch):
    cid = lax.axis_index("core")
    sid = lax.axis_index("subcore")

    @pl.when(sid < 4)
    def _():
        _sc_batch(cid * 4 + sid, mx_hbm, distr_hbm, nllr_hbm, out_hbm,
                  *scratch)


def _sc_select(mxr, distr, nllr):
    mesh = plsc.VectorSubcoreMesh(core_axis_name="core",
                                  subcore_axis_name="subcore",
                                  num_cores=2, num_subcores=16)
    return pl.kernel(
        _sc_body,
        out_type=jax.ShapeDtypeStruct((8, 16), jnp.float32),
        mesh=mesh,
        compiler_params=pltpu.CompilerParams(needs_layout_passes=False),
        scratch_types=[
            pltpu.VMEM((16384,), jnp.float32),
            pltpu.VMEM((1024,), jnp.float32),
            pltpu.VMEM((1024,), jnp.int32),
            pltpu.VMEM((1024,), jnp.int32),
            pltpu.VMEM((512, 128), jnp.float32),
            pltpu.VMEM((16384,), jnp.int32),
            pltpu.VMEM((16384,), jnp.int32),
            pltpu.VMEM((1024,), jnp.int32),
            pltpu.VMEM((1024,), jnp.int32),
            pltpu.VMEM((1024,), jnp.int32),
            pltpu.VMEM((1024,), jnp.int32),
            pltpu.VMEM((16,), jnp.float32),
            pltpu.SemaphoreType.DMA,
        ],
    )(mxr, distr, nllr)


def kernel(inputs, targets):
    B, C, H, W = inputs.shape
    grid1 = (B, H // _RB)
    dist, nll, mx = pl.pallas_call(
        _stage1_body,
        grid=grid1,
        in_specs=[
            pl.BlockSpec((1, C, _RB, W), lambda b, i: (b, 0, i, 0)),
            pl.BlockSpec((1, _RB, W), lambda b, i: (b, i, 0)),
        ],
        out_specs=[
            pl.BlockSpec((1, _RB, W), lambda b, i: (b, i, 0)),
            pl.BlockSpec((1, _RB, W), lambda b, i: (b, i, 0)),
            pl.BlockSpec((1, _RB, 32), lambda b, i: (b, i, 0)),
        ],
        out_shape=[
            jax.ShapeDtypeStruct((B, H, W), jnp.float32),
            jax.ShapeDtypeStruct((B, H, W), jnp.float32),
            jax.ShapeDtypeStruct((B, H, 32), jnp.float32),
        ],
    )(inputs, targets)

    nrow128 = B * H * 4
    out = _sc_select(mx.reshape(B, H * 32),
                     dist.reshape(nrow128, 128),
                     nll.reshape(nrow128, 128))
    s_gt, s_eq, m, e = out[:, 0], out[:, 1], out[:, 2], out[:, 3]
    contrib = s_gt + (_KEEP - m) * s_eq / e
    return jnp.sum(contrib) / (B * _KEEP)


# speedup vs baseline: 1.4041x; 1.0836x over previous
import functools

import jax
import jax.numpy as jnp
from jax import lax
from jax.experimental import pallas as pl
from jax.experimental.pallas import tpu as pltpu
from jax.experimental.pallas import tpu_sc as plsc

_C = 19
_KEEP = 512
_RB = 128
_PINF = 0x7F800000


def _stage1_body(inp_ref, tgt_ref, dist_ref, nll_ref, mx_ref):
    t = tgt_ref[0]
    x = inp_ref[0, 0]
    s_exp = jnp.exp(x)
    s_abs = jnp.abs(x)
    xt = jnp.where(t == 0, x, 0.0)
    for c in range(1, _C):
        x = inp_ref[0, c]
        s_exp = s_exp + jnp.exp(x)
        s_abs = s_abs + jnp.abs(x)
        xt = jnp.where(t == c, x, xt)
    dist = s_abs - jnp.abs(xt) + jnp.abs(xt - 1.0)
    dist_ref[0] = dist
    nll_ref[0] = jnp.log(s_exp) - xt
    mx_ref[0] = jnp.max(dist.reshape(_RB, 32, 16), axis=-1)


def _sc_batch(b, mx_hbm, distr_hbm, nllr_hbm, out_hbm, mx_v, l3_v,
              rowidx_v, r128idx_v, rows512_v, ebits_v, eidx_v, gr128_v,
              goff_v, qr128_v, qoff_v, outb_v, dsem):
    iota = lax.iota(jnp.int32, 16)
    zeros_i = jnp.zeros((16,), jnp.int32)

    def _zero(ref, n):
        @pl.loop(0, n, step=128)
        def _(i):
            for u in range(8):
                ref[pl.ds(i + u * 16, 16)] = zeros_i

    _zero(ebits_v, 16384)
    _zero(rowidx_v, 1024)
    _zero(r128idx_v, 1024)
    _zero(gr128_v, 1024)
    _zero(goff_v, 1024)
    _zero(qr128_v, 1024)
    _zero(qoff_v, 1024)

    pltpu.sync_copy(mx_hbm.at[b], mx_v)

    @pl.loop(0, 1024, step=16)
    def _(j):
        idx0 = j * 16 + iota * 16
        acc = plsc.load_gather(mx_v, [idx0])
        for k in range(1, 16):
            acc = jnp.maximum(acc, plsc.load_gather(mx_v, [idx0 + k]))
        l3_v[pl.ds(j, 16)] = acc

    def _count_gt(ref, is_f32, nslice128, mid):
        def step(i, cv):
            for u in range(8):
                v = ref[pl.ds(i * 128 + u * 16, 16)]
                bits = plsc.bitcast(v, jnp.int32) if is_f32 else v
                cv = cv + (bits > mid).astype(jnp.int32)
            return cv

        cv = lax.fori_loop(0, nslice128, step, jnp.zeros((16,), jnp.int32))
        return jnp.sum(cv, axis=0)

    def _kth_bits(ref, is_f32, nslice128, lo0, hi0):
        def cond(lohi):
            lo, hi = lohi
            return hi - lo > 1

        def bis(lohi):
            lo, hi = lohi
            mid = lo + (hi - lo) // 2
            below = _count_gt(ref, is_f32, nslice128, mid) < _KEEP
            return (jnp.where(below, lo, mid), jnp.where(below, mid, hi))

        _, hi = lax.while_loop(cond, bis, (lo0, hi0))
        return hi

    def mmstep(i, mv):
        return jnp.maximum(mv, l3_v[pl.ds(i * 16, 16)])

    mxall = lax.fori_loop(0, 64, mmstep, jnp.zeros((16,), jnp.float32))
    maxbits = jnp.max(plsc.bitcast(mxall, jnp.int32), axis=0)

    f3 = _kth_bits(l3_v, True, jnp.int32(8), jnp.int32(-1), maxbits + 1)

    def cstep(i, ptr):
        for u in range(4):
            bits = plsc.bitcast(mx_v[pl.ds(i * 64 + u * 16, 16)], jnp.int32)
            mask = bits >= f3
            plsc.store_compressed(ebits_v.at[pl.ds(ptr, 16)], bits, mask=mask)
            c = plsc.all_reduce_population_count(mask)
            ptr = jnp.minimum(ptr + c[0], 16384 - 16)
        return ptr

    ncand = lax.fori_loop(0, 256, cstep, jnp.int32(0))
    f2 = _kth_bits(ebits_v, False, (ncand + 127) // 128, f3 - 1,
                   maxbits + 1)

    _zero(ebits_v, 16384)

    rowbase = b * 16384

    def rstep(i, ptr):
        for u in range(4):
            bits = plsc.bitcast(mx_v[pl.ds(i * 64 + u * 16, 16)], jnp.int32)
            mask = bits >= f2
            gids = rowbase + i * 64 + u * 16 + iota
            plsc.store_compressed(rowidx_v.at[pl.ds(ptr, 16)], gids,
                                  mask=mask)
            plsc.store_compressed(r128idx_v.at[pl.ds(ptr, 16)], gids >> 3,
                                  mask=mask)
            c = plsc.all_reduce_population_count(mask)
            ptr = jnp.minimum(ptr + c[0], 1024 - 16)
        return ptr

    rowcnt = lax.fori_loop(0, 256, rstep, jnp.int32(0))

    copies = [
        pltpu.async_copy(
            distr_hbm.at[r128idx_v.at[pl.ds(k * 128, 128)]],
            rows512_v.at[pl.ds(k * 128, 128), :], dsem)
        for k in range(4)
    ]
    for cp in copies:
        cp.wait()
    rowlim = jnp.minimum(rowcnt, 512)

    def estep(j, eptr):
        for t in range(16):
            rloc = j * 16 + t
            rv = plsc.load_gather(rowidx_v, [jnp.full((16,), rloc)])
            cols = (rv & 7) * 16 + iota
            vals = plsc.load_gather(rows512_v,
                                    [jnp.full((16,), rloc), cols])
            bits = plsc.bitcast(vals, jnp.int32)
            mask = (bits >= f2) & (rloc < rowlim)
            plsc.store_compressed(ebits_v.at[pl.ds(eptr, 16)], bits,
                                  mask=mask)
            plsc.store_compressed(eidx_v.at[pl.ds(eptr, 16)],
                                  rv * 16 + iota, mask=mask)
            c = plsc.all_reduce_population_count(mask)
            eptr = jnp.minimum(eptr + c[0], 16384 - 16)
        return eptr

    ecnt = lax.fori_loop(0, (rowlim + 15) // 16, estep, jnp.int32(0))

    thr = _kth_bits(ebits_v, False, (ecnt + 127) // 128, f2 - 1,
                    maxbits + 1)

    def sstep(i, carry):
        m, e, gptr, qptr = carry
        bits = ebits_v[pl.ds(i * 16, 16)]
        eidx = eidx_v[pl.ds(i * 16, 16)]
        gt = bits > thr
        eq = bits == thr
        r128v = eidx >> 7
        offv = eidx & 127
        plsc.store_compressed(gr128_v.at[pl.ds(gptr, 16)], r128v, mask=gt)
        plsc.store_compressed(goff_v.at[pl.ds(gptr, 16)], offv, mask=gt)
        plsc.store_compressed(qr128_v.at[pl.ds(qptr, 16)], r128v, mask=eq)
        plsc.store_compressed(qoff_v.at[pl.ds(qptr, 16)], offv, mask=eq)
        gc = plsc.all_reduce_population_count(gt)
        qc = plsc.all_reduce_population_count(eq)
        return (m + gc[0], e + qc[0],
                jnp.minimum(gptr + gc[0], 1024 - 16),
                jnp.minimum(qptr + qc[0], 1024 - 16))

    z = jnp.int32(0)
    m, e, _, _ = lax.fori_loop(0, (ecnt + 15) // 16, sstep, (z, z, z, z))

    def _gather_sum(r128_ref, off_ref, n, nchunk):
        copies = [
            pltpu.async_copy(
                nllr_hbm.at[r128_ref.at[pl.ds(k * 128, 128)]],
                rows512_v.at[pl.ds(k * 128, 128), :], dsem)
            for k in range(nchunk)
        ]
        for cp in copies:
            cp.wait()

        def ss(i, acc):
            pos = i * 16 + iota
            offs = off_ref[pl.ds(i * 16, 16)]
            vals = plsc.load_gather(rows512_v, [pos, offs])
            return acc + jnp.where(pos < n, vals, 0.0)

        acc = lax.fori_loop(0, (n + 15) // 16, ss,
                            jnp.zeros((16,), jnp.float32))
        return jnp.sum(acc, axis=0)

    s_gt = _gather_sum(gr128_v, goff_v, m, 4)
    e = jnp.minimum(e, 128)
    s_eq = _gather_sum(qr128_v, qoff_v, e, 1)

    stats = jnp.where(iota == 0, s_gt, 0.0)
    stats = jnp.where(iota == 1, s_eq, stats)
    stats = jnp.where(iota == 2, m.astype(jnp.float32), stats)
    stats = jnp.where(iota == 3, e.astype(jnp.float32), stats)
    outb_v[...] = stats
    pltpu.sync_copy(outb_v, out_hbm.at[b])


def _sc_body(mx_hbm, distr_hbm, nllr_hbm, out_hbm, *scratch):
    cid = lax.axis_index("core")
    sid = lax.axis_index("subcore")

    @pl.when(sid < 4)
    def _():
        _sc_batch(cid * 4 + sid, mx_hbm, distr_hbm, nllr_hbm, out_hbm,
                  *scratch)


def _sc_select(mxr, distr, nllr):
    mesh = plsc.VectorSubcoreMesh(core_axis_name="core",
                                  subcore_axis_name="subcore",
                                  num_cores=2, num_subcores=16)
    return pl.kernel(
        _sc_body,
        out_type=jax.ShapeDtypeStruct((8, 16), jnp.float32),
        mesh=mesh,
        compiler_params=pltpu.CompilerParams(needs_layout_passes=False),
        scratch_types=[
            pltpu.VMEM((16384,), jnp.float32),
            pltpu.VMEM((1024,), jnp.float32),
            pltpu.VMEM((1024,), jnp.int32),
            pltpu.VMEM((1024,), jnp.int32),
            pltpu.VMEM((512, 128), jnp.float32),
            pltpu.VMEM((16384,), jnp.int32),
            pltpu.VMEM((16384,), jnp.int32),
            pltpu.VMEM((1024,), jnp.int32),
            pltpu.VMEM((1024,), jnp.int32),
            pltpu.VMEM((1024,), jnp.int32),
            pltpu.VMEM((1024,), jnp.int32),
            pltpu.VMEM((16,), jnp.float32),
            pltpu.SemaphoreType.DMA,
        ],
    )(mxr, distr, nllr)


def kernel(inputs, targets):
    B, C, H, W = inputs.shape
    grid1 = (B, H // _RB)
    dist, nll, mx = pl.pallas_call(
        _stage1_body,
        grid=grid1,
        in_specs=[
            pl.BlockSpec((1, C, _RB, W), lambda b, i: (b, 0, i, 0)),
            pl.BlockSpec((1, _RB, W), lambda b, i: (b, i, 0)),
        ],
        out_specs=[
            pl.BlockSpec((1, _RB, W), lambda b, i: (b, i, 0)),
            pl.BlockSpec((1, _RB, W), lambda b, i: (b, i, 0)),
            pl.BlockSpec((1, _RB, 32), lambda b, i: (b, i, 0)),
        ],
        out_shape=[
            jax.ShapeDtypeStruct((B, H, W), jnp.float32),
            jax.ShapeDtypeStruct((B, H, W), jnp.float32),
            jax.ShapeDtypeStruct((B, H, 32), jnp.float32),
        ],
    )(inputs, targets)

    nrow128 = B * H * 4
    out = _sc_select(mx.reshape(B, H * 32),
                     dist.reshape(nrow128, 128),
                     nll.reshape(nrow128, 128))
    s_gt, s_eq, m, e = out[:, 0], out[:, 1], out[:, 2], out[:, 3]
    contrib = s_gt + (_KEEP - m) * s_eq / e
    return jnp.sum(contrib) / (B * _KEEP)


# speedup vs baseline: 1.4510x; 1.0334x over previous
import functools

import jax
import jax.numpy as jnp
from jax import lax
from jax.experimental import pallas as pl
from jax.experimental.pallas import tpu as pltpu
from jax.experimental.pallas import tpu_sc as plsc

_C = 19
_KEEP = 512
_RB = 256
_PINF = 0x7F800000


def _stage1_body(inp_ref, tgt_ref, dist_ref, nll_ref, mx_ref):
    t = tgt_ref[0]
    x = inp_ref[0, 0]
    s_exp = jnp.exp(x)
    s_abs = jnp.abs(x)
    xt = jnp.where(t == 0, x, 0.0)
    for c in range(1, _C):
        x = inp_ref[0, c]
        s_exp = s_exp + jnp.exp(x)
        s_abs = s_abs + jnp.abs(x)
        xt = jnp.where(t == c, x, xt)
    dist = s_abs - jnp.abs(xt) + jnp.abs(xt - 1.0)
    dist_ref[0] = dist
    nll_ref[0] = jnp.log(s_exp) - xt
    mx_ref[0] = jnp.max(dist.reshape(_RB, 32, 16), axis=-1)


def _sc_batch(b, mx_hbm, distr_hbm, nllr_hbm, out_hbm, mx_v, l3_v,
              rowidx_v, r128idx_v, rows512_v, ebits_v, eidx_v, gr128_v,
              goff_v, qr128_v, qoff_v, outb_v, dsem):
    iota = lax.iota(jnp.int32, 16)
    zeros_i = jnp.zeros((16,), jnp.int32)

    def _zero(ref, n):
        @pl.loop(0, n, step=128)
        def _(i):
            for u in range(8):
                ref[pl.ds(i + u * 16, 16)] = zeros_i

    _zero(ebits_v, 16384)
    _zero(rowidx_v, 1024)
    _zero(r128idx_v, 1024)
    _zero(gr128_v, 1024)
    _zero(goff_v, 1024)
    _zero(qr128_v, 1024)
    _zero(qoff_v, 1024)

    pltpu.sync_copy(mx_hbm.at[b], mx_v)

    @pl.loop(0, 1024, step=16)
    def _(j):
        idx0 = j * 16 + iota * 16
        acc = plsc.load_gather(mx_v, [idx0])
        for k in range(1, 16):
            acc = jnp.maximum(acc, plsc.load_gather(mx_v, [idx0 + k]))
        l3_v[pl.ds(j, 16)] = acc

    def _count_gt(ref, is_f32, nslice128, mid):
        def step(i, cv):
            for u in range(8):
                v = ref[pl.ds(i * 128 + u * 16, 16)]
                bits = plsc.bitcast(v, jnp.int32) if is_f32 else v
                cv = cv + (bits > mid).astype(jnp.int32)
            return cv

        cv = lax.fori_loop(0, nslice128, step, jnp.zeros((16,), jnp.int32))
        return jnp.sum(cv, axis=0)

    def _kth_bits(ref, is_f32, nslice128, lo0, hi0):
        def cond(lohi):
            lo, hi = lohi
            return hi - lo > 1

        def bis(lohi):
            lo, hi = lohi
            mid = lo + (hi - lo) // 2
            below = _count_gt(ref, is_f32, nslice128, mid) < _KEEP
            return (jnp.where(below, lo, mid), jnp.where(below, mid, hi))

        _, hi = lax.while_loop(cond, bis, (lo0, hi0))
        return hi

    def mmstep(i, mv):
        return jnp.maximum(mv, l3_v[pl.ds(i * 16, 16)])

    mxall = lax.fori_loop(0, 64, mmstep, jnp.zeros((16,), jnp.float32))
    maxbits = jnp.max(plsc.bitcast(mxall, jnp.int32), axis=0)

    f3 = _kth_bits(l3_v, True, jnp.int32(8), jnp.int32(-1), maxbits + 1)

    def cstep(i, ptr):
        for u in range(4):
            bits = plsc.bitcast(mx_v[pl.ds(i * 64 + u * 16, 16)], jnp.int32)
            mask = bits >= f3
            plsc.store_compressed(ebits_v.at[pl.ds(ptr, 16)], bits, mask=mask)
            c = plsc.all_reduce_population_count(mask)
            ptr = jnp.minimum(ptr + c[0], 16384 - 16)
        return ptr

    ncand = lax.fori_loop(0, 256, cstep, jnp.int32(0))
    f2 = _kth_bits(ebits_v, False, (ncand + 127) // 128, f3 - 1,
                   maxbits + 1)

    _zero(ebits_v, 16384)

    rowbase = b * 16384

    def rstep(i, ptr):
        for u in range(4):
            bits = plsc.bitcast(mx_v[pl.ds(i * 64 + u * 16, 16)], jnp.int32)
            mask = bits >= f2
            gids = rowbase + i * 64 + u * 16 + iota
            plsc.store_compressed(rowidx_v.at[pl.ds(ptr, 16)], gids,
                                  mask=mask)
            plsc.store_compressed(r128idx_v.at[pl.ds(ptr, 16)], gids >> 3,
                                  mask=mask)
            c = plsc.all_reduce_population_count(mask)
            ptr = jnp.minimum(ptr + c[0], 1024 - 16)
        return ptr

    rowcnt = lax.fori_loop(0, 256, rstep, jnp.int32(0))

    copies = [
        pltpu.async_copy(
            distr_hbm.at[r128idx_v.at[pl.ds(k * 128, 128)]],
            rows512_v.at[pl.ds(k * 128, 128), :], dsem)
        for k in range(4)
    ]
    for cp in copies:
        cp.wait()
    rowlim = jnp.minimum(rowcnt, 512)

    def estep(j, eptr):
        for t in range(16):
            rloc = j * 16 + t
            rv = plsc.load_gather(rowidx_v, [jnp.full((16,), rloc)])
            cols = (rv & 7) * 16 + iota
            vals = plsc.load_gather(rows512_v,
                                    [jnp.full((16,), rloc), cols])
            bits = plsc.bitcast(vals, jnp.int32)
            mask = (bits >= f2) & (rloc < rowlim)
            plsc.store_compressed(ebits_v.at[pl.ds(eptr, 16)], bits,
                                  mask=mask)
            plsc.store_compressed(eidx_v.at[pl.ds(eptr, 16)],
                                  rv * 16 + iota, mask=mask)
            c = plsc.all_reduce_population_count(mask)
            eptr = jnp.minimum(eptr + c[0], 16384 - 16)
        return eptr

    ecnt = lax.fori_loop(0, (rowlim + 15) // 16, estep, jnp.int32(0))

    thr = _kth_bits(ebits_v, False, (ecnt + 127) // 128, f2 - 1,
                    maxbits + 1)

    def sstep(i, carry):
        m, e, gptr, qptr = carry
        bits = ebits_v[pl.ds(i * 16, 16)]
        eidx = eidx_v[pl.ds(i * 16, 16)]
        gt = bits > thr
        eq = bits == thr
        r128v = eidx >> 7
        offv = eidx & 127
        plsc.store_compressed(gr128_v.at[pl.ds(gptr, 16)], r128v, mask=gt)
        plsc.store_compressed(goff_v.at[pl.ds(gptr, 16)], offv, mask=gt)
        plsc.store_compressed(qr128_v.at[pl.ds(qptr, 16)], r128v, mask=eq)
        plsc.store_compressed(qoff_v.at[pl.ds(qptr, 16)], offv, mask=eq)
        gc = plsc.all_reduce_population_count(gt)
        qc = plsc.all_reduce_population_count(eq)
        return (m + gc[0], e + qc[0],
                jnp.minimum(gptr + gc[0], 1024 - 16),
                jnp.minimum(qptr + qc[0], 1024 - 16))

    z = jnp.int32(0)
    m, e, _, _ = lax.fori_loop(0, (ecnt + 15) // 16, sstep, (z, z, z, z))

    def _gather_sum(r128_ref, off_ref, n, nchunk):
        copies = [
            pltpu.async_copy(
                nllr_hbm.at[r128_ref.at[pl.ds(k * 128, 128)]],
                rows512_v.at[pl.ds(k * 128, 128), :], dsem)
            for k in range(nchunk)
        ]
        for cp in copies:
            cp.wait()

        def ss(i, acc):
            pos = i * 16 + iota
            offs = off_ref[pl.ds(i * 16, 16)]
            vals = plsc.load_gather(rows512_v, [pos, offs])
            return acc + jnp.where(pos < n, vals, 0.0)

        acc = lax.fori_loop(0, (n + 15) // 16, ss,
                            jnp.zeros((16,), jnp.float32))
        return jnp.sum(acc, axis=0)

    s_gt = _gather_sum(gr128_v, goff_v, m, 4)
    e = jnp.minimum(e, 128)
    s_eq = _gather_sum(qr128_v, qoff_v, e, 1)

    stats = jnp.where(iota == 0, s_gt, 0.0)
    stats = jnp.where(iota == 1, s_eq, stats)
    stats = jnp.where(iota == 2, m.astype(jnp.float32), stats)
    stats = jnp.where(iota == 3, e.astype(jnp.float32), stats)
    outb_v[...] = stats
    pltpu.sync_copy(outb_v, out_hbm.at[b])


def _sc_body(mx_hbm, distr_hbm, nllr_hbm, out_hbm, *scratch):
    cid = lax.axis_index("core")
    sid = lax.axis_index("subcore")

    @pl.when(sid < 4)
    def _():
        _sc_batch(cid * 4 + sid, mx_hbm, distr_hbm, nllr_hbm, out_hbm,
                  *scratch)


def _sc_select(mxr, distr, nllr):
    mesh = plsc.VectorSubcoreMesh(core_axis_name="core",
                                  subcore_axis_name="subcore",
                                  num_cores=2, num_subcores=16)
    return pl.kernel(
        _sc_body,
        out_type=jax.ShapeDtypeStruct((8, 16), jnp.float32),
        mesh=mesh,
        compiler_params=pltpu.CompilerParams(needs_layout_passes=False),
        scratch_types=[
            pltpu.VMEM((16384,), jnp.float32),
            pltpu.VMEM((1024,), jnp.float32),
            pltpu.VMEM((1024,), jnp.int32),
            pltpu.VMEM((1024,), jnp.int32),
            pltpu.VMEM((512, 128), jnp.float32),
            pltpu.VMEM((16384,), jnp.int32),
            pltpu.VMEM((16384,), jnp.int32),
            pltpu.VMEM((1024,), jnp.int32),
            pltpu.VMEM((1024,), jnp.int32),
            pltpu.VMEM((1024,), jnp.int32),
            pltpu.VMEM((1024,), jnp.int32),
            pltpu.VMEM((16,), jnp.float32),
            pltpu.SemaphoreType.DMA,
        ],
    )(mxr, distr, nllr)


def kernel(inputs, targets):
    B, C, H, W = inputs.shape
    grid1 = (B, H // _RB)
    dist, nll, mx = pl.pallas_call(
        _stage1_body,
        grid=grid1,
        in_specs=[
            pl.BlockSpec((1, C, _RB, W), lambda b, i: (b, 0, i, 0)),
            pl.BlockSpec((1, _RB, W), lambda b, i: (b, i, 0)),
        ],
        out_specs=[
            pl.BlockSpec((1, _RB, W), lambda b, i: (b, i, 0)),
            pl.BlockSpec((1, _RB, W), lambda b, i: (b, i, 0)),
            pl.BlockSpec((1, _RB, 32), lambda b, i: (b, i, 0)),
        ],
        out_shape=[
            jax.ShapeDtypeStruct((B, H, W), jnp.float32),
            jax.ShapeDtypeStruct((B, H, W), jnp.float32),
            jax.ShapeDtypeStruct((B, H, 32), jnp.float32),
        ],
    )(inputs, targets)

    nrow128 = B * H * 4
    out = _sc_select(mx.reshape(B, H * 32),
                     dist.reshape(nrow128, 128),
                     nll.reshape(nrow128, 128))
    s_gt, s_eq, m, e = out[:, 0], out[:, 1], out[:, 2], out[:, 3]
    contrib = s_gt + (_KEEP - m) * s_eq / e
    return jnp.sum(contrib) / (B * _KEEP)
